# Initial kernel scaffold; baseline (speedup 1.0000x reference)
import jax
import jax.numpy as jnp
from jax.experimental import pallas as pl


def kernel(attn, table):
    raise NotImplementedError("write your pallas kernel here")


# speedup vs baseline: 123.3050x; 123.3050x over previous
import jax
import jax.numpy as jnp
from jax.experimental import pallas as pl
from jax.experimental.pallas import tpu as pltpu

_H = 12
_T = 2048
_SPAN = 2048
_VW = 4096
_PAD = 128
_CT_W = 4352
_R = 512
_SUB = 128


def _bias_add_kernel(tab_ref, attn_ref, out_ref, v_ref):
    q = pl.program_id(1)

    @pl.when(q == 0)
    def _build_skewed_table():
        for r in range(_SUB):
            v_ref[r, :] = tab_ref[0, 0, pl.ds(_PAD - r, _VW)]

    for s in range(_R // _SUB):
        row = _SUB * s
        start = _SUB * ((_SPAN // _SUB) - (_R // _SUB) * q - s)
        out_ref[0, pl.ds(row, _SUB), :] = (
            attn_ref[0, pl.ds(row, _SUB), :] + v_ref[:, pl.ds(start, _T)]
        )


def kernel(attn, table):
    h, b, t, l = attn.shape
    ct = jnp.zeros((_H, 1, _CT_W), dtype=attn.dtype)
    ct = jax.lax.dynamic_update_slice(
        ct, table.T.reshape(_H, 1, 2 * _SPAN + 1), (0, 0, _PAD))

    attn3 = attn.reshape(_H, t, l)
    nq = t // _R
    out3 = pl.pallas_call(
        _bias_add_kernel,
        grid=(_H, nq),
        in_specs=[
            pl.BlockSpec((1, 1, _CT_W), lambda hh, qq: (hh, 0, 0)),
            pl.BlockSpec((1, _R, _T), lambda hh, qq: (hh, qq, 0)),
        ],
        out_specs=pl.BlockSpec((1, _R, _T), lambda hh, qq: (hh, qq, 0)),
        out_shape=jax.ShapeDtypeStruct((_H, t, l), attn.dtype),
        scratch_shapes=[pltpu.VMEM((_SUB, _VW), jnp.float32)],
        compiler_params=pltpu.CompilerParams(
            dimension_semantics=("parallel", "arbitrary"),
        ),
    )(ct, attn3)
    return out3.reshape(attn.shape)


# speedup vs baseline: 126.9741x; 1.0298x over previous
import jax
import jax.numpy as jnp
from jax.experimental import pallas as pl
from jax.experimental.pallas import tpu as pltpu

_H = 12
_T = 2048
_SPAN = 2048
_VW = 4096
_PAD = 128
_CT_W = 4352
_R = 1024
_SUB = 128


def _bias_add_kernel(tab_ref, attn_ref, out_ref, v_ref):
    q = pl.program_id(1)

    @pl.when(q == 0)
    def _build_skewed_table():
        for r in range(_SUB):
            v_ref[r, :] = tab_ref[0, 0, pl.ds(_PAD - r, _VW)]

    for s in range(_R // _SUB):
        row = _SUB * s
        start = _SUB * ((_SPAN // _SUB) - (_R // _SUB) * q - s)
        out_ref[0, pl.ds(row, _SUB), :] = (
            attn_ref[0, pl.ds(row, _SUB), :] + v_ref[:, pl.ds(start, _T)]
        )


def kernel(attn, table):
    h, b, t, l = attn.shape
    ct = jnp.zeros((_H, 1, _CT_W), dtype=attn.dtype)
    ct = jax.lax.dynamic_update_slice(
        ct, table.T.reshape(_H, 1, 2 * _SPAN + 1), (0, 0, _PAD))

    attn3 = attn.reshape(_H, t, l)
    nq = t // _R
    out3 = pl.pallas_call(
        _bias_add_kernel,
        grid=(_H, nq),
        in_specs=[
            pl.BlockSpec((1, 1, _CT_W), lambda hh, qq: (hh, 0, 0)),
            pl.BlockSpec((1, _R, _T), lambda hh, qq: (hh, qq, 0)),
        ],
        out_specs=pl.BlockSpec((1, _R, _T), lambda hh, qq: (hh, qq, 0)),
        out_shape=jax.ShapeDtypeStruct((_H, t, l), attn.dtype),
        scratch_shapes=[pltpu.VMEM((_SUB, _VW), jnp.float32)],
        compiler_params=pltpu.CompilerParams(
            dimension_semantics=("parallel", "arbitrary"),
        ),
    )(ct, attn3)
    return out3.reshape(attn.shape)
